# baseline (speedup 1.0000x reference)
import functools

import jax
import jax.numpy as jnp
from jax import lax
from jax.experimental import pallas as pl
from jax.experimental.pallas import tpu as pltpu
from jax.experimental.pallas import tpu_sc as plsc

_MAXLEN = 8192
_DIM = 1024
_NC = 2
_ROWS_PER_SC = _MAXLEN // _NC
_CHUNK = 512
_NCHUNK = _ROWS_PER_SC // _CHUNK

_mesh = plsc.ScalarSubcoreMesh(axis_name="c", num_cores=_NC)


@functools.partial(
    pl.kernel,
    mesh=_mesh,
    out_type=jax.ShapeDtypeStruct((_MAXLEN, _DIM), jnp.float32),
    scratch_types=[
        pltpu.VMEM_SHARED((2, _CHUNK, _DIM), jnp.float32),
        pltpu.SemaphoreType.DMA,
        pltpu.SemaphoreType.DMA,
    ],
)
def _pe_copy(table_hbm, out_hbm, buf, isem, osem):
    cid = lax.axis_index("c")
    base = cid * _ROWS_PER_SC

    ins = []
    outs = []

    def start_in(c):
        h = pltpu.make_async_copy(
            table_hbm.at[pl.ds(base + c * _CHUNK, _CHUNK)], buf.at[c % 2], isem)
        h.start()
        ins.append(h)

    def start_out(c):
        h = pltpu.make_async_copy(
            buf.at[c % 2],
            out_hbm.at[pl.ds(base + c * _CHUNK, _CHUNK)], osem)
        h.start()
        outs.append(h)

    start_in(0)
    for c in range(_NCHUNK):
        if c + 1 < _NCHUNK:
            if c >= 1:
                outs[c - 1].wait()
            start_in(c + 1)
        ins[c].wait()
        start_out(c)
    outs[_NCHUNK - 2].wait()
    outs[_NCHUNK - 1].wait()


def kernel(length, emb):
    del length
    out = _pe_copy(emb)
    return out[None, :, :]
